# baseline (device time: 47972 ns/iter reference)
import jax
import jax.numpy as jnp
from jax import lax
from jax.experimental import pallas as pl
from jax.experimental.pallas import tpu as pltpu

N_DEV = 8
S = 1024
H = 128
E = S + 2 * H
HQ = 8
DH = 128
SCALE = 0.08838834764831843


def kernel(x, Wq, K_ext, V_ext, Wo):
    def body(x_ref, wq_ref, k_ref, v_ref, wo_ref, out_ref,
             kf_ref, vf_ref, send_sems, recv_sems):
        my = lax.axis_index("i")
        left = lax.rem(my + N_DEV - 1, N_DEV)
        right = lax.rem(my + 1, N_DEV)

        barrier_sem = pltpu.get_barrier_semaphore()
        for nbr in (left, right):
            pl.semaphore_signal(
                barrier_sem, inc=1,
                device_id=(nbr,), device_id_type=pl.DeviceIdType.MESH,
            )
        pl.semaphore_wait(barrier_sem, 2)

        kf_ref[pl.ds(H, S)] = k_ref[0]
        vf_ref[pl.ds(H, S)] = v_ref[0]

        transfers = (
            (k_ref.at[0, pl.ds(0, H)], kf_ref.at[pl.ds(H + S, H)], left, 0),
            (k_ref.at[0, pl.ds(S - H, H)], kf_ref.at[pl.ds(0, H)], right, 1),
            (v_ref.at[0, pl.ds(0, H)], vf_ref.at[pl.ds(H + S, H)], left, 2),
            (v_ref.at[0, pl.ds(S - H, H)], vf_ref.at[pl.ds(0, H)], right, 3),
        )
        rdmas = []
        for src, dst, dev, idx in transfers:
            r = pltpu.make_async_remote_copy(
                src_ref=src, dst_ref=dst,
                send_sem=send_sems.at[idx], recv_sem=recv_sems.at[idx],
                device_id=(dev,), device_id_type=pl.DeviceIdType.MESH,
            )
            r.start()
            rdmas.append(r)

        q = jnp.dot(x_ref[0], wq_ref[...],
                    preferred_element_type=jnp.float32)

        for r in rdmas:
            r.wait()

        kf = kf_ref[...].reshape(E, HQ * DH)
        vf = vf_ref[...].reshape(E, HQ * DH)

        i_idx = lax.broadcasted_iota(jnp.int32, (S, E), 0)
        j_idx = lax.broadcasted_iota(jnp.int32, (S, E), 1)
        gk = my * S - H + j_idx
        mask = (j_idx >= i_idx) & (j_idx <= i_idx + 2 * H) \
            & (gk >= 0) & (gk < N_DEV * S)
        neg = jnp.float32(-1e9)

        ctx_parts = []
        for h in range(HQ):
            sl = slice(h * DH, (h + 1) * DH)
            s = lax.dot_general(
                q[:, sl], kf[:, sl], (((1,), (1,)), ((), ())),
                preferred_element_type=jnp.float32,
            ) * SCALE
            s = jnp.where(mask, s, neg)
            m = jnp.max(s, axis=1, keepdims=True)
            w = jnp.exp(s - m)
            w = w / jnp.sum(w, axis=1, keepdims=True)
            ctx_parts.append(
                jnp.dot(w, vf[:, sl], preferred_element_type=jnp.float32))
        ctx = jnp.concatenate(ctx_parts, axis=1)
        out_ref[0] = jnp.dot(ctx, wo_ref[...],
                             preferred_element_type=jnp.float32)

    return pl.pallas_call(
        body,
        out_shape=jax.ShapeDtypeStruct((1, S, HQ * DH), jnp.float32),
        in_specs=[pl.BlockSpec(memory_space=pltpu.VMEM)] * 5,
        out_specs=pl.BlockSpec(memory_space=pltpu.VMEM),
        scratch_shapes=[
            pltpu.VMEM((E, HQ, DH), jnp.float32),
            pltpu.VMEM((E, HQ, DH), jnp.float32),
            pltpu.SemaphoreType.DMA((4,)),
            pltpu.SemaphoreType.DMA((4,)),
        ],
        compiler_params=pltpu.CompilerParams(
            collective_id=0,
            vmem_limit_bytes=100 * 1024 * 1024,
        ),
    )(x, Wq, K_ext, V_ext, Wo)


# device time: 35713 ns/iter; 1.3433x vs baseline; 1.3433x over previous
import jax
import jax.numpy as jnp
from jax import lax
from jax.experimental import pallas as pl
from jax.experimental.pallas import tpu as pltpu

N_DEV = 8
S = 1024
H = 128
E = S + 2 * H
HQ = 8
DH = 128
B = 256
C = B + 2 * H
SCALE = 0.08838834764831843


def kernel(x, Wq, K_ext, V_ext, Wo):
    def body(x_ref, wq_ref, k_ref, v_ref, wo_ref, out_ref,
             kf_ref, vf_ref, send_sems, recv_sems):
        my = lax.axis_index("i")
        left = lax.rem(my + N_DEV - 1, N_DEV)
        right = lax.rem(my + 1, N_DEV)

        barrier_sem = pltpu.get_barrier_semaphore()
        for nbr in (left, right):
            pl.semaphore_signal(
                barrier_sem, inc=1,
                device_id=(nbr,), device_id_type=pl.DeviceIdType.MESH,
            )
        pl.semaphore_wait(barrier_sem, 2)

        transfers = (
            (k_ref.at[0, pl.ds(0, H)], kf_ref.at[pl.ds(H + S, H)], left, 0),
            (k_ref.at[0, pl.ds(S - H, H)], kf_ref.at[pl.ds(0, H)], right, 1),
            (v_ref.at[0, pl.ds(0, H)], vf_ref.at[pl.ds(H + S, H)], left, 2),
            (v_ref.at[0, pl.ds(S - H, H)], vf_ref.at[pl.ds(0, H)], right, 3),
        )
        rdmas = []
        for src, dst, dev, idx in transfers:
            r = pltpu.make_async_remote_copy(
                src_ref=src, dst_ref=dst,
                send_sem=send_sems.at[idx], recv_sem=recv_sems.at[idx],
                device_id=(dev,), device_id_type=pl.DeviceIdType.MESH,
            )
            r.start()
            rdmas.append(r)

        kf_ref[pl.ds(H, S)] = k_ref[0]
        vf_ref[pl.ds(H, S)] = v_ref[0]
        q = jnp.dot(x_ref[0], wq_ref[...],
                    preferred_element_type=jnp.float32)

        i_loc = lax.broadcasted_iota(jnp.int32, (B, C), 0)
        j_loc = lax.broadcasted_iota(jnp.int32, (B, C), 1)
        band = (j_loc >= i_loc) & (j_loc <= i_loc + 2 * H)

        def do_block(b):
            c0 = b * B
            gk0 = my * S - H + c0
            mask = band & (gk0 + j_loc >= 0) & (gk0 + j_loc < N_DEV * S)
            kfb = kf_ref[pl.ds(c0, C)].reshape(C, HQ * DH)
            vfb = vf_ref[pl.ds(c0, C)].reshape(C, HQ * DH)
            ctx_parts = []
            for h in range(HQ):
                sl = slice(h * DH, (h + 1) * DH)
                s = lax.dot_general(
                    q[c0:c0 + B, sl], kfb[:, sl],
                    (((1,), (1,)), ((), ())),
                    preferred_element_type=jnp.float32,
                ) * SCALE
                w = jnp.where(mask, jnp.exp(s), 0.0)
                denom = jnp.sum(w, axis=1, keepdims=True)
                ctx_parts.append(
                    jnp.dot(w, vfb[:, sl],
                            preferred_element_type=jnp.float32) / denom)
            ctx = jnp.concatenate(ctx_parts, axis=1)
            out_ref[0, pl.ds(c0, B)] = jnp.dot(
                ctx, wo_ref[...], preferred_element_type=jnp.float32)

        do_block(1)
        do_block(2)
        rdmas[1].wait_recv()
        rdmas[3].wait_recv()
        do_block(0)
        rdmas[0].wait_recv()
        rdmas[2].wait_recv()
        do_block(3)
        for r in rdmas:
            r.wait_send()

    return pl.pallas_call(
        body,
        out_shape=jax.ShapeDtypeStruct((1, S, HQ * DH), jnp.float32),
        in_specs=[pl.BlockSpec(memory_space=pltpu.VMEM)] * 5,
        out_specs=pl.BlockSpec(memory_space=pltpu.VMEM),
        scratch_shapes=[
            pltpu.VMEM((E, HQ, DH), jnp.float32),
            pltpu.VMEM((E, HQ, DH), jnp.float32),
            pltpu.SemaphoreType.DMA((4,)),
            pltpu.SemaphoreType.DMA((4,)),
        ],
        compiler_params=pltpu.CompilerParams(
            collective_id=0,
            vmem_limit_bytes=100 * 1024 * 1024,
        ),
    )(x, Wq, K_ext, V_ext, Wo)


# device time: 33987 ns/iter; 1.4115x vs baseline; 1.0508x over previous
import jax
import jax.numpy as jnp
from jax import lax
from jax.experimental import pallas as pl
from jax.experimental.pallas import tpu as pltpu

N_DEV = 8
S = 1024
H = 128
E = S + 2 * H
HQ = 8
DH = 128
B = 256
C = B + 2 * H
SCALE = 0.08838834764831843


def kernel(x, Wq, K_ext, V_ext, Wo):
    def body(x_ref, wq_ref, k_ref, v_ref, wo_ref, out_ref,
             kf_ref, vf_ref, send_sems, recv_sems):
        my = lax.axis_index("i")
        left = lax.rem(my + N_DEV - 1, N_DEV)
        right = lax.rem(my + 1, N_DEV)

        barrier_sem = pltpu.get_barrier_semaphore()
        for nbr in (left, right):
            pl.semaphore_signal(
                barrier_sem, inc=1,
                device_id=(nbr,), device_id_type=pl.DeviceIdType.MESH,
            )
        pl.semaphore_wait(barrier_sem, 2)

        kf_ref[pl.ds(H, S)] = k_ref[0].reshape(S, HQ * DH).astype(jnp.bfloat16)
        vf_ref[pl.ds(H, S)] = v_ref[0].reshape(S, HQ * DH).astype(jnp.bfloat16)

        transfers = (
            (kf_ref.at[pl.ds(H, H)], kf_ref.at[pl.ds(H + S, H)], left, 0),
            (kf_ref.at[pl.ds(S, H)], kf_ref.at[pl.ds(0, H)], right, 1),
            (vf_ref.at[pl.ds(H, H)], vf_ref.at[pl.ds(H + S, H)], left, 2),
            (vf_ref.at[pl.ds(S, H)], vf_ref.at[pl.ds(0, H)], right, 3),
        )
        rdmas = []
        for src, dst, dev, idx in transfers:
            r = pltpu.make_async_remote_copy(
                src_ref=src, dst_ref=dst,
                send_sem=send_sems.at[idx], recv_sem=recv_sems.at[idx],
                device_id=(dev,), device_id_type=pl.DeviceIdType.MESH,
            )
            r.start()
            rdmas.append(r)

        q = jnp.dot(x_ref[0], wq_ref[...],
                    preferred_element_type=jnp.float32)
        q16 = q.astype(jnp.bfloat16)

        i_loc = lax.broadcasted_iota(jnp.int32, (B, C), 0)
        j_loc = lax.broadcasted_iota(jnp.int32, (B, C), 1)
        band = (j_loc >= i_loc) & (j_loc <= i_loc + 2 * H)

        def do_block(b):
            c0 = b * B
            gk0 = my * S - H + c0
            mask = band & (gk0 + j_loc >= 0) & (gk0 + j_loc < N_DEV * S)
            kfb = kf_ref[pl.ds(c0, C)]
            vfb = vf_ref[pl.ds(c0, C)]
            ctx_parts = []
            for h in range(HQ):
                sl = slice(h * DH, (h + 1) * DH)
                s = lax.dot_general(
                    q16[c0:c0 + B, sl], kfb[:, sl],
                    (((1,), (1,)), ((), ())),
                    preferred_element_type=jnp.float32,
                ) * SCALE
                w = jnp.where(mask, jnp.exp(s), 0.0)
                denom = jnp.sum(w, axis=1, keepdims=True)
                ctx_parts.append(
                    jnp.dot(w.astype(jnp.bfloat16), vfb[:, sl],
                            preferred_element_type=jnp.float32) / denom)
            ctx = jnp.concatenate(ctx_parts, axis=1)
            out_ref[0, pl.ds(c0, B)] = jnp.dot(
                ctx, wo_ref[...], preferred_element_type=jnp.float32)

        do_block(1)
        do_block(2)
        rdmas[1].wait_recv()
        rdmas[3].wait_recv()
        do_block(0)
        rdmas[0].wait_recv()
        rdmas[2].wait_recv()
        do_block(3)
        for r in rdmas:
            r.wait_send()

    return pl.pallas_call(
        body,
        out_shape=jax.ShapeDtypeStruct((1, S, HQ * DH), jnp.float32),
        in_specs=[pl.BlockSpec(memory_space=pltpu.VMEM)] * 5,
        out_specs=pl.BlockSpec(memory_space=pltpu.VMEM),
        scratch_shapes=[
            pltpu.VMEM((E, HQ * DH), jnp.bfloat16),
            pltpu.VMEM((E, HQ * DH), jnp.bfloat16),
            pltpu.SemaphoreType.DMA((4,)),
            pltpu.SemaphoreType.DMA((4,)),
        ],
        compiler_params=pltpu.CompilerParams(
            collective_id=0,
            vmem_limit_bytes=100 * 1024 * 1024,
        ),
    )(x, Wq, K_ext, V_ext, Wo)


# device time: 30325 ns/iter; 1.5819x vs baseline; 1.1208x over previous
import jax
import jax.numpy as jnp
from jax import lax
from jax.experimental import pallas as pl
from jax.experimental.pallas import tpu as pltpu

N_DEV = 8
S = 1024
H = 128
E = S + 2 * H
HQ = 8
DH = 128
B = 256
C = B + 2 * H
SCALE = 0.08838834764831843


def kernel(x, Wq, K_ext, V_ext, Wo):
    def body(x_hbm, wq_hbm, k_hbm, v_hbm, wo_hbm, out_hbm,
             xv_ref, wqv_ref, wov_ref, kf_ref, vf_ref, octx_ref,
             copy_sems, out_sems, send_sems, recv_sems):
        my = lax.axis_index("i")
        left = lax.rem(my + N_DEV - 1, N_DEV)
        right = lax.rem(my + 1, N_DEV)

        cp_x = pltpu.make_async_copy(x_hbm.at[0], xv_ref, copy_sems.at[0])
        cp_wq = pltpu.make_async_copy(wq_hbm, wqv_ref, copy_sems.at[1])
        cp_wo = pltpu.make_async_copy(wo_hbm, wov_ref, copy_sems.at[2])
        cp_k = pltpu.make_async_copy(
            k_hbm.at[0], kf_ref.at[pl.ds(H, S)], copy_sems.at[3])
        cp_v = pltpu.make_async_copy(
            v_hbm.at[0], vf_ref.at[pl.ds(H, S)], copy_sems.at[4])
        for cp in (cp_x, cp_wq, cp_wo, cp_k, cp_v):
            cp.start()

        barrier_sem = pltpu.get_barrier_semaphore()
        for nbr in (left, right):
            pl.semaphore_signal(
                barrier_sem, inc=1,
                device_id=(nbr,), device_id_type=pl.DeviceIdType.MESH,
            )
        pl.semaphore_wait(barrier_sem, 2)

        transfers = (
            (k_hbm.at[0, pl.ds(0, H)], kf_ref.at[pl.ds(H + S, H)], left, 0),
            (k_hbm.at[0, pl.ds(S - H, H)], kf_ref.at[pl.ds(0, H)], right, 1),
            (v_hbm.at[0, pl.ds(0, H)], vf_ref.at[pl.ds(H + S, H)], left, 2),
            (v_hbm.at[0, pl.ds(S - H, H)], vf_ref.at[pl.ds(0, H)], right, 3),
        )
        rdmas = []
        for src, dst, dev, idx in transfers:
            r = pltpu.make_async_remote_copy(
                src_ref=src, dst_ref=dst,
                send_sem=send_sems.at[idx], recv_sem=recv_sems.at[idx],
                device_id=(dev,), device_id_type=pl.DeviceIdType.MESH,
            )
            r.start()
            rdmas.append(r)

        cp_x.wait()
        cp_wq.wait()
        q = jnp.dot(xv_ref[...], wqv_ref[...],
                    preferred_element_type=jnp.float32)
        q16 = q.astype(jnp.bfloat16)

        i_loc = lax.broadcasted_iota(jnp.int32, (B, C), 0)
        j_loc = lax.broadcasted_iota(jnp.int32, (B, C), 1)
        band = (j_loc >= i_loc) & (j_loc <= i_loc + 2 * H)

        def do_block(b):
            c0 = b * B
            gk0 = my * S - H + c0
            mask = band & (gk0 + j_loc >= 0) & (gk0 + j_loc < N_DEV * S)
            kfb = kf_ref[pl.ds(c0, C)].reshape(C, HQ * DH).astype(jnp.bfloat16)
            vfb = vf_ref[pl.ds(c0, C)].reshape(C, HQ * DH).astype(jnp.bfloat16)
            ctx_parts = []
            for h in range(HQ):
                sl = slice(h * DH, (h + 1) * DH)
                s = lax.dot_general(
                    q16[c0:c0 + B, sl], kfb[:, sl],
                    (((1,), (1,)), ((), ())),
                    preferred_element_type=jnp.float32,
                ) * SCALE
                w = jnp.where(mask, jnp.exp(s), 0.0)
                denom = jnp.sum(w, axis=1, keepdims=True)
                ctx_parts.append(
                    jnp.dot(w.astype(jnp.bfloat16), vfb[:, sl],
                            preferred_element_type=jnp.float32) / denom)
            ctx = jnp.concatenate(ctx_parts, axis=1)
            octx_ref[pl.ds(c0, B)] = jnp.dot(
                ctx, wov_ref[...], preferred_element_type=jnp.float32)
            cp_out = pltpu.make_async_copy(
                octx_ref.at[pl.ds(c0, B)], out_hbm.at[0, pl.ds(c0, B)],
                out_sems.at[b])
            cp_out.start()
            return cp_out

        cp_wo.wait()
        cp_k.wait()
        cp_v.wait()
        outs = [None] * 4
        outs[1] = do_block(1)
        outs[2] = do_block(2)
        rdmas[1].wait_recv()
        rdmas[3].wait_recv()
        outs[0] = do_block(0)
        rdmas[0].wait_recv()
        rdmas[2].wait_recv()
        outs[3] = do_block(3)
        for r in rdmas:
            r.wait_send()
        for cp in outs:
            cp.wait()

    return pl.pallas_call(
        body,
        out_shape=jax.ShapeDtypeStruct((1, S, HQ * DH), jnp.float32),
        in_specs=[pl.BlockSpec(memory_space=pl.ANY)] * 5,
        out_specs=pl.BlockSpec(memory_space=pl.ANY),
        scratch_shapes=[
            pltpu.VMEM((S, HQ * DH), jnp.float32),
            pltpu.VMEM((HQ * DH, HQ * DH), jnp.float32),
            pltpu.VMEM((HQ * DH, HQ * DH), jnp.float32),
            pltpu.VMEM((E, HQ, DH), jnp.float32),
            pltpu.VMEM((E, HQ, DH), jnp.float32),
            pltpu.VMEM((S, HQ * DH), jnp.float32),
            pltpu.SemaphoreType.DMA((5,)),
            pltpu.SemaphoreType.DMA((4,)),
            pltpu.SemaphoreType.DMA((4,)),
            pltpu.SemaphoreType.DMA((4,)),
        ],
        compiler_params=pltpu.CompilerParams(
            collective_id=0,
            vmem_limit_bytes=100 * 1024 * 1024,
        ),
    )(x, Wq, K_ext, V_ext, Wo)


# device time: 26596 ns/iter; 1.8037x vs baseline; 1.1402x over previous
import jax
import jax.numpy as jnp
from jax import lax
from jax.experimental import pallas as pl
from jax.experimental.pallas import tpu as pltpu

N_DEV = 8
S = 1024
H = 128
E = S + 2 * H
HQ = 8
DH = 128
B = 256
C = B + 2 * H
SCALE = 0.08838834764831843


def kernel(x, Wq, K_ext, V_ext, Wo):
    def body(x_hbm, wq_hbm, k_hbm, v_hbm, wo_hbm, out_hbm,
             xv_ref, wqv_ref, wov_ref, kf_ref, vf_ref, octx_ref,
             copy_sems, out_sems, send_sems, recv_sems):
        my = lax.axis_index("i")
        left = lax.rem(my + N_DEV - 1, N_DEV)
        right = lax.rem(my + 1, N_DEV)
        has_left = my > 0
        has_right = my < N_DEV - 1

        cp_x = pltpu.make_async_copy(x_hbm.at[0], xv_ref, copy_sems.at[0])
        cp_wq = pltpu.make_async_copy(wq_hbm, wqv_ref, copy_sems.at[1])
        cp_x.start()
        cp_wq.start()
        cp_k = pltpu.make_async_copy(
            k_hbm.at[0], kf_ref.at[pl.ds(H, S)], copy_sems.at[3])
        cp_v = pltpu.make_async_copy(
            v_hbm.at[0], vf_ref.at[pl.ds(H, S)], copy_sems.at[4])
        cp_wo = pltpu.make_async_copy(wo_hbm, wov_ref, copy_sems.at[2])
        cp_k.start()
        cp_v.start()
        cp_wo.start()

        barrier_sem = pltpu.get_barrier_semaphore()

        @pl.when(has_left)
        def _():
            pl.semaphore_signal(
                barrier_sem, inc=1,
                device_id=(left,), device_id_type=pl.DeviceIdType.MESH)

        @pl.when(has_right)
        def _():
            pl.semaphore_signal(
                barrier_sem, inc=1,
                device_id=(right,), device_id_type=pl.DeviceIdType.MESH)

        n_nbrs = has_left.astype(jnp.int32) + has_right.astype(jnp.int32)
        pl.semaphore_wait(barrier_sem, n_nbrs)

        def left_rdmas():
            return (
                pltpu.make_async_remote_copy(
                    src_ref=k_hbm.at[0, pl.ds(0, H)],
                    dst_ref=kf_ref.at[pl.ds(H + S, H)],
                    send_sem=send_sems.at[0], recv_sem=recv_sems.at[0],
                    device_id=(left,), device_id_type=pl.DeviceIdType.MESH),
                pltpu.make_async_remote_copy(
                    src_ref=v_hbm.at[0, pl.ds(0, H)],
                    dst_ref=vf_ref.at[pl.ds(H + S, H)],
                    send_sem=send_sems.at[2], recv_sem=recv_sems.at[2],
                    device_id=(left,), device_id_type=pl.DeviceIdType.MESH),
            )

        def right_rdmas():
            return (
                pltpu.make_async_remote_copy(
                    src_ref=k_hbm.at[0, pl.ds(S - H, H)],
                    dst_ref=kf_ref.at[pl.ds(0, H)],
                    send_sem=send_sems.at[1], recv_sem=recv_sems.at[1],
                    device_id=(right,), device_id_type=pl.DeviceIdType.MESH),
                pltpu.make_async_remote_copy(
                    src_ref=v_hbm.at[0, pl.ds(S - H, H)],
                    dst_ref=vf_ref.at[pl.ds(0, H)],
                    send_sem=send_sems.at[3], recv_sem=recv_sems.at[3],
                    device_id=(right,), device_id_type=pl.DeviceIdType.MESH),
            )

        @pl.when(has_left)
        def _():
            for r in left_rdmas():
                r.start()

        @pl.when(has_right)
        def _():
            for r in right_rdmas():
                r.start()

        @pl.when(jnp.logical_not(has_left))
        def _():
            kf_ref[pl.ds(0, H)] = jnp.zeros((H, HQ, DH), jnp.float32)
            vf_ref[pl.ds(0, H)] = jnp.zeros((H, HQ, DH), jnp.float32)

        @pl.when(jnp.logical_not(has_right))
        def _():
            kf_ref[pl.ds(H + S, H)] = jnp.zeros((H, HQ, DH), jnp.float32)
            vf_ref[pl.ds(H + S, H)] = jnp.zeros((H, HQ, DH), jnp.float32)

        cp_x.wait()
        cp_wq.wait()
        q = jnp.dot(xv_ref[...], wqv_ref[...],
                    preferred_element_type=jnp.float32)
        q16 = q.astype(jnp.bfloat16)

        i_loc = lax.broadcasted_iota(jnp.int32, (B, C), 0)
        j_loc = lax.broadcasted_iota(jnp.int32, (B, C), 1)
        band = (j_loc >= i_loc) & (j_loc <= i_loc + 2 * H)

        def do_block(b):
            c0 = b * B
            gk0 = my * S - H + c0
            mask = band & (gk0 + j_loc >= 0) & (gk0 + j_loc < N_DEV * S)
            kfb = kf_ref[pl.ds(c0, C)].reshape(C, HQ * DH).astype(jnp.bfloat16)
            vfb = vf_ref[pl.ds(c0, C)].reshape(C, HQ * DH).astype(jnp.bfloat16)
            ctx_parts = []
            for h in range(HQ):
                sl = slice(h * DH, (h + 1) * DH)
                s = lax.dot_general(
                    q16[c0:c0 + B, sl], kfb[:, sl],
                    (((1,), (1,)), ((), ())),
                    preferred_element_type=jnp.float32,
                ) * SCALE
                w = jnp.where(mask, jnp.exp(s), 0.0)
                denom = jnp.sum(w, axis=1, keepdims=True)
                ctx_parts.append(
                    jnp.dot(w.astype(jnp.bfloat16), vfb[:, sl],
                            preferred_element_type=jnp.float32) / denom)
            ctx = jnp.concatenate(ctx_parts, axis=1)
            octx_ref[pl.ds(c0, B)] = jnp.dot(
                ctx, wov_ref[...], preferred_element_type=jnp.float32)
            cp_out = pltpu.make_async_copy(
                octx_ref.at[pl.ds(c0, B)], out_hbm.at[0, pl.ds(c0, B)],
                out_sems.at[b])
            cp_out.start()
            return cp_out

        cp_wo.wait()
        cp_k.wait()
        cp_v.wait()
        outs = [None] * 4
        outs[1] = do_block(1)
        outs[2] = do_block(2)

        @pl.when(has_left)
        def _():
            for r in right_rdmas():
                r.wait_recv()

        outs[0] = do_block(0)

        @pl.when(has_right)
        def _():
            for r in left_rdmas():
                r.wait_recv()

        outs[3] = do_block(3)

        @pl.when(has_left)
        def _():
            for r in left_rdmas():
                r.wait_send()

        @pl.when(has_right)
        def _():
            for r in right_rdmas():
                r.wait_send()

        for cp in outs:
            cp.wait()

    return pl.pallas_call(
        body,
        out_shape=jax.ShapeDtypeStruct((1, S, HQ * DH), jnp.float32),
        in_specs=[pl.BlockSpec(memory_space=pl.ANY)] * 5,
        out_specs=pl.BlockSpec(memory_space=pl.ANY),
        scratch_shapes=[
            pltpu.VMEM((S, HQ * DH), jnp.float32),
            pltpu.VMEM((HQ * DH, HQ * DH), jnp.float32),
            pltpu.VMEM((HQ * DH, HQ * DH), jnp.float32),
            pltpu.VMEM((E, HQ, DH), jnp.float32),
            pltpu.VMEM((E, HQ, DH), jnp.float32),
            pltpu.VMEM((S, HQ * DH), jnp.float32),
            pltpu.SemaphoreType.DMA((5,)),
            pltpu.SemaphoreType.DMA((4,)),
            pltpu.SemaphoreType.DMA((4,)),
            pltpu.SemaphoreType.DMA((4,)),
        ],
        compiler_params=pltpu.CompilerParams(
            collective_id=0,
            vmem_limit_bytes=100 * 1024 * 1024,
        ),
    )(x, Wq, K_ext, V_ext, Wo)
